# baseline (speedup 1.0000x reference)
import functools

import jax
import jax.numpy as jnp
from jax import lax
from jax.experimental import pallas as pl
from jax.experimental.pallas import tpu as pltpu
from jax.experimental.pallas import tpu_sc as plsc

N = 10000
NHE = 10000
NI = 320000
DIN = 128
DH = 256
DOUT = 128
EPS = 1e-5
KEEP = 0.7

NPAD = 10240
DUMMY = 10000
NC = 2
NS = 16
NW = NC * NS
CHUNK = 128
GD = 80
TOTCH = NW * GD
NIPAD = TOTCH * CHUNK
G0 = 80
G1 = TOTCH // NS - G0
GMAX = max(G0, G1)
IDXROWS = TOTCH + GMAX
RPT = NPAD // NS
KSLAB = RPT // CHUNK

_f32 = jnp.float32


def _zero_rows_buf(buf):
    zeros16 = jnp.zeros((16,), _f32)

    def body(r, carry):
        for l in range(DIN // 16):
            buf[r, pl.ds(l * 16, 16)] = zeros16
        return carry

    lax.fori_loop(0, CHUNK, body, 0)


def _zero_vec(ref):
    zeros16 = jnp.zeros((16,), _f32)

    def body(i, carry):
        ref[pl.ds(i * 16, 16)] = zeros16
        return carry

    lax.fori_loop(0, NPAD // 16, body, 0)


def _deg_body(nidx_hbm, hidx_hbm, eattr_hbm,
              dp_hbm, bp_hbm, cp_hbm,
              nidx_v, hidx_v, eattr_v, dloc, bloc, cloc):
    c = lax.axis_index("c")
    s = lax.axis_index("s")
    w = s * NC + c
    pltpu.sync_copy(nidx_hbm.at[pl.ds(w * GD, GD)], nidx_v)
    pltpu.sync_copy(hidx_hbm.at[pl.ds(w * GD, GD)], hidx_v)
    pltpu.sync_copy(eattr_hbm, eattr_v)
    _zero_vec(dloc)
    _zero_vec(bloc)
    _zero_vec(cloc)
    ones16 = jnp.ones((16,), _f32)

    def chunk(g, carry):
        for j in range(CHUNK // 16):
            ni = nidx_v[g, pl.ds(j * 16, 16)]
            hi = hidx_v[g, pl.ds(j * 16, 16)]
            wv = plsc.load_gather(eattr_v, [hi])
            plsc.addupdate_scatter(dloc, [ni], wv)
            plsc.addupdate_scatter(bloc, [hi], ones16)
            plsc.addupdate_scatter(cloc, [ni], ones16)
        return carry

    lax.fori_loop(0, GD, chunk, 0)
    pltpu.sync_copy(dloc, dp_hbm.at[w])
    pltpu.sync_copy(bloc, bp_hbm.at[w])
    pltpu.sync_copy(cloc, cp_hbm.at[w])


HALF = GMAX // 2


def _sc_prop_body(src_hbm, gidx_hbm, sidx_hbm, acc_hbm,
                  gidx_v, sidx_v, rows2, acc_sh, gsem):
    c = lax.axis_index("c")
    s = lax.axis_index("s")
    base = (s * NC + c) * GMAX
    _zero_rows_buf(rows2.at[0])
    dz = [pltpu.async_copy(rows2.at[0],
                           acc_sh.at[pl.ds(s * RPT + k * CHUNK, CHUNK)],
                           gsem)
          for k in range(KSLAB)]
    for d in dz:
        d.wait()
    plsc.subcore_barrier()

    for h in range(2):
        pltpu.sync_copy(gidx_hbm.at[pl.ds(base + h * HALF, HALF)], gidx_v)
        pltpu.sync_copy(sidx_hbm.at[pl.ds(base + h * HALF, HALF)], sidx_v)
        pltpu.async_copy(src_hbm.at[gidx_v.at[0]], rows2.at[0], gsem)

        def chunk(g, carry):
            par = lax.rem(g, 2)
            nxt = 1 - par
            gn = jnp.minimum(g + 1, HALF - 1)
            pltpu.async_copy(src_hbm.at[gidx_v.at[gn]], rows2.at[nxt], gsem)
            pltpu.make_async_copy(src_hbm.at[gidx_v.at[g]], rows2.at[par],
                                  gsem).wait()
            pltpu.sync_copy(rows2.at[par], acc_sh.at[sidx_v.at[g]], add=True)
            return carry

        lax.fori_loop(0, HALF, chunk, 0)
        pltpu.make_async_copy(src_hbm.at[gidx_v.at[HALF - 1]],
                              rows2.at[0], gsem).wait()
    plsc.subcore_barrier()
    dw = [pltpu.async_copy(acc_sh.at[pl.ds(s * RPT + k * CHUNK, CHUNK)],
                           acc_hbm.at[c, pl.ds(s * RPT + k * CHUNK, CHUNK)],
                           gsem)
          for k in range(KSLAB)]
    for d in dw:
        d.wait()


def _sc_mesh():
    return plsc.VectorSubcoreMesh(core_axis_name="c", subcore_axis_name="s")


def _sc_degrees(nidx, hidx, eattr):
    return pl.kernel(
        _deg_body,
        compiler_params=pltpu.CompilerParams(needs_layout_passes=False),
        out_type=(jax.ShapeDtypeStruct((NW, NPAD), _f32),
                  jax.ShapeDtypeStruct((NW, NPAD), _f32),
                  jax.ShapeDtypeStruct((NW, NPAD), _f32)),
        mesh=_sc_mesh(),
        scratch_types=[
            pltpu.VMEM((GD, CHUNK), jnp.int32),
            pltpu.VMEM((GD, CHUNK), jnp.int32),
            pltpu.VMEM((NPAD,), _f32),
            pltpu.VMEM((NPAD,), _f32),
            pltpu.VMEM((NPAD,), _f32),
            pltpu.VMEM((NPAD,), _f32),
        ],
    )(nidx, hidx, eattr)


def _sc_prop(src, gidx, sidx):
    return pl.kernel(
        _sc_prop_body,
        compiler_params=pltpu.CompilerParams(needs_layout_passes=False),
        out_type=jax.ShapeDtypeStruct((NC, NPAD, DIN), _f32),
        mesh=_sc_mesh(),
        scratch_types=[
            pltpu.VMEM((HALF, CHUNK), jnp.int32),
            pltpu.VMEM((HALF, CHUNK), jnp.int32),
            pltpu.VMEM((2, CHUNK, DIN), _f32),
            pltpu.VMEM_SHARED((NPAD, DIN), _f32),
            pltpu.SemaphoreType.DMA,
        ],
    )(src, gidx, sidx)


def _tca_body(accp, dp, bp, cp, oute, binv, dinv, sval):
    i = pl.program_id(0)
    ones = jnp.ones((NW, 1), _f32)
    dn = (((0,), (0,)), ((), ()))
    dsum = lax.dot_general(dp[...], ones, dn, preferred_element_type=_f32)
    bsum = lax.dot_general(bp[...], ones, dn, preferred_element_type=_f32)
    csum = lax.dot_general(cp[...], ones, dn, preferred_element_type=_f32)
    rowid = i * 128 + lax.broadcasted_iota(jnp.int32, (128, 1), 0)
    valid = rowid < N
    bi = jnp.where(valid & (bsum > 0), 1.0 / bsum, 0.0)
    di = jnp.where(valid & (dsum > 0), 1.0 / dsum, 0.0)
    oute[...] = bi * (accp[0] + accp[1])
    binv[...] = bi
    dinv[...] = di
    sval[...] = csum * di


def _tc_combine_a(accp, dp, bp, cp):
    return pl.pallas_call(
        _tca_body,
        grid=(NPAD // 128,),
        in_specs=[
            pl.BlockSpec((2, 128, DIN), lambda i: (0, i, 0)),
            pl.BlockSpec((NW, 128), lambda i: (0, i)),
            pl.BlockSpec((NW, 128), lambda i: (0, i)),
            pl.BlockSpec((NW, 128), lambda i: (0, i)),
        ],
        out_specs=[
            pl.BlockSpec((128, DIN), lambda i: (i, 0)),
            pl.BlockSpec((128, 1), lambda i: (i, 0)),
            pl.BlockSpec((128, 1), lambda i: (i, 0)),
            pl.BlockSpec((128, 1), lambda i: (i, 0)),
        ],
        out_shape=[
            jax.ShapeDtypeStruct((NPAD, DIN), _f32),
            jax.ShapeDtypeStruct((NPAD, 1), _f32),
            jax.ShapeDtypeStruct((NPAD, 1), _f32),
            jax.ShapeDtypeStruct((NPAD, 1), _f32),
        ],
    )(accp, dp, bp, cp)


def _tcc_body(accp, binv, oute):
    oute[...] = binv[...] * (accp[0] + accp[1])


def _tc_combine_c(accp, binv):
    return pl.pallas_call(
        _tcc_body,
        grid=(NPAD // 128,),
        in_specs=[
            pl.BlockSpec((2, 128, DIN), lambda i: (0, i, 0)),
            pl.BlockSpec((128, 1), lambda i: (i, 0)),
        ],
        out_specs=pl.BlockSpec((128, DIN), lambda i: (i, 0)),
        out_shape=jax.ShapeDtypeStruct((NPAD, DIN), _f32),
    )(accp, binv)


def _tcb_body(accp, dinv, sval, mask, w1, b1, w2, b2, zout):
    xp = dinv[...] * (accp[0] + accp[1])
    dn = (((1,), (1,)), ((), ()))
    pre = lax.dot_general(xp, w1[...], dn, preferred_element_type=_f32)
    pre = pre + sval[...] * b1[...]
    h = jnp.maximum(pre, 0.0) * mask[...]
    z = lax.dot_general(h, w2[...], dn, preferred_element_type=_f32) + b2[...]
    zout[...] = z


def _tc_mlp(accp, dinv, sval, mask, w1, b1, w2, b2):
    return pl.pallas_call(
        _tcb_body,
        grid=(NPAD // 128,),
        in_specs=[
            pl.BlockSpec((2, 128, DIN), lambda i: (0, i, 0)),
            pl.BlockSpec((128, 1), lambda i: (i, 0)),
            pl.BlockSpec((128, 1), lambda i: (i, 0)),
            pl.BlockSpec((128, DH), lambda i: (i, 0)),
            pl.BlockSpec((DH, DIN), lambda i: (0, 0)),
            pl.BlockSpec((1, DH), lambda i: (0, 0)),
            pl.BlockSpec((DOUT, DH), lambda i: (0, 0)),
            pl.BlockSpec((1, DOUT), lambda i: (0, 0)),
        ],
        out_specs=pl.BlockSpec((128, DOUT), lambda i: (i, 0)),
        out_shape=jax.ShapeDtypeStruct((NPAD, DOUT), _f32),
    )(accp, dinv, sval, mask, w1, b1, w2, b2)


def _tcd_body(accp, dinv, gamma, beta, yout):
    v = dinv[...] * (accp[0] + accp[1])
    mu = jnp.mean(v, axis=1, keepdims=True)
    d = v - mu
    var = jnp.mean(d * d, axis=1, keepdims=True)
    yout[...] = d * lax.rsqrt(var + EPS) * gamma[...] + beta[...]


def _tc_layernorm(accp, dinv, gamma, beta):
    return pl.pallas_call(
        _tcd_body,
        grid=(NPAD // 128,),
        in_specs=[
            pl.BlockSpec((2, 128, DOUT), lambda i: (0, i, 0)),
            pl.BlockSpec((128, 1), lambda i: (i, 0)),
            pl.BlockSpec((1, DOUT), lambda i: (0, 0)),
            pl.BlockSpec((1, DOUT), lambda i: (0, 0)),
        ],
        out_specs=pl.BlockSpec((128, DOUT), lambda i: (i, 0)),
        out_shape=jax.ShapeDtypeStruct((NPAD, DOUT), _f32),
    )(accp, dinv, gamma, beta)


def kernel(x, edge_index, edge_attr, W1, b1, W2, b2, gamma, beta):
    xpad = jnp.zeros((NPAD, DIN), _f32).at[:N].set(x)
    eattr = jnp.zeros((NPAD,), _f32).at[:NHE].set(edge_attr)
    npadinc = IDXROWS * CHUNK - NI
    pad = DUMMY + (jnp.arange(npadinc, dtype=jnp.int32) % (NPAD - N))
    nidx = jnp.concatenate([edge_index[0], pad]).reshape(IDXROWS, CHUNK)
    hidx = jnp.concatenate([edge_index[1], pad]).reshape(IDXROWS, CHUNK)
    keep = jax.random.bernoulli(jax.random.key(42), KEEP, (N, DH))
    mask = jnp.zeros((NPAD, DH), _f32).at[:N].set(
        jnp.where(keep, 1.0 / KEEP, 0.0))

    dpart, bpart, cpart = _sc_degrees(nidx, hidx, eattr)
    acc_e = _sc_prop(xpad, nidx, hidx)
    out_e, binv, dinv, sval = _tc_combine_a(acc_e, dpart, bpart, cpart)
    acc_n = _sc_prop(out_e, hidx, nidx)
    z = _tc_mlp(acc_n, dinv, sval, mask, W1, b1.reshape(1, DH),
                W2, b2.reshape(1, DOUT))

    acc_e2 = _sc_prop(z, nidx, hidx)
    out_e2 = _tc_combine_c(acc_e2, binv)
    acc_n2 = _sc_prop(out_e2, hidx, nidx)
    y = _tc_layernorm(acc_n2, dinv, gamma.reshape(1, DOUT),
                      beta.reshape(1, DOUT))
    return y[:N]


# speedup vs baseline: 1.2331x; 1.2331x over previous
import functools

import jax
import jax.numpy as jnp
from jax import lax
from jax.experimental import pallas as pl
from jax.experimental.pallas import tpu as pltpu
from jax.experimental.pallas import tpu_sc as plsc

N = 10000
NHE = 10000
NI = 320000
DIN = 128
DH = 256
DOUT = 128
EPS = 1e-5
KEEP = 0.7

NPAD = 10240
DUMMY = 10000
NC = 2
NS = 16
NW = NC * NS
CHUNK = 128
GD = 80
TOTCH = NW * GD
NIPAD = TOTCH * CHUNK
G0 = 80
G1 = TOTCH // NS - G0
GMAX = max(G0, G1)
IDXROWS = TOTCH + GMAX
RPT = NPAD // NS
KSLAB = RPT // CHUNK

_f32 = jnp.float32


def _zero_rows_buf(buf):
    zeros16 = jnp.zeros((16,), _f32)

    def body(r, carry):
        for l in range(DIN // 16):
            buf[r, pl.ds(l * 16, 16)] = zeros16
        return carry

    lax.fori_loop(0, CHUNK, body, 0)


def _zero_vec(ref):
    zeros16 = jnp.zeros((16,), _f32)

    def body(i, carry):
        ref[pl.ds(i * 16, 16)] = zeros16
        return carry

    lax.fori_loop(0, NPAD // 16, body, 0)


def _deg_body(nidx_hbm, hidx_hbm, eattr_hbm,
              dp_hbm, bp_hbm, cp_hbm,
              nidx_v, hidx_v, eattr_v, dloc, bloc, cloc):
    c = lax.axis_index("c")
    s = lax.axis_index("s")
    w = s * NC + c
    pltpu.sync_copy(nidx_hbm.at[pl.ds(w * GD, GD)], nidx_v)
    pltpu.sync_copy(hidx_hbm.at[pl.ds(w * GD, GD)], hidx_v)
    pltpu.sync_copy(eattr_hbm, eattr_v)
    _zero_vec(dloc)
    _zero_vec(bloc)
    _zero_vec(cloc)
    ones16 = jnp.ones((16,), _f32)

    def chunk(g, carry):
        for j in range(CHUNK // 16):
            ni = nidx_v[g, pl.ds(j * 16, 16)]
            hi = hidx_v[g, pl.ds(j * 16, 16)]
            wv = plsc.load_gather(eattr_v, [hi])
            plsc.addupdate_scatter(dloc, [ni], wv)
            plsc.addupdate_scatter(bloc, [hi], ones16)
            plsc.addupdate_scatter(cloc, [ni], ones16)
        return carry

    lax.fori_loop(0, GD, chunk, 0)
    pltpu.sync_copy(dloc, dp_hbm.at[w])
    pltpu.sync_copy(bloc, bp_hbm.at[w])
    pltpu.sync_copy(cloc, cp_hbm.at[w])


HALF = GMAX // 2


def _sc_prop_body(src_hbm, gidx_hbm, sidx_hbm, acc_hbm,
                  gidx_v, sidx_v, rows2, acc_sh, gsem):
    c = lax.axis_index("c")
    s = lax.axis_index("s")
    base = (s * NC + c) * GMAX
    _zero_rows_buf(rows2.at[0])
    dz = [pltpu.async_copy(rows2.at[0],
                           acc_sh.at[pl.ds(s * RPT + k * CHUNK, CHUNK)],
                           gsem)
          for k in range(KSLAB)]
    for d in dz:
        d.wait()
    plsc.subcore_barrier()

    for h in range(2):
        pltpu.sync_copy(gidx_hbm.at[pl.ds(base + h * HALF, HALF)], gidx_v)
        pltpu.sync_copy(sidx_hbm.at[pl.ds(base + h * HALF, HALF)], sidx_v)
        pltpu.async_copy(src_hbm.at[gidx_v.at[0]], rows2.at[0], gsem)

        def chunk(g, carry):
            par = lax.rem(g, 2)
            nxt = 1 - par
            gn = jnp.minimum(g + 1, HALF - 1)
            pltpu.async_copy(src_hbm.at[gidx_v.at[gn]], rows2.at[nxt], gsem)
            pltpu.make_async_copy(src_hbm.at[gidx_v.at[g]], rows2.at[par],
                                  gsem).wait()
            pltpu.sync_copy(rows2.at[par], acc_sh.at[sidx_v.at[g]], add=True)
            return carry

        lax.fori_loop(0, HALF, chunk, 0)
        pltpu.make_async_copy(src_hbm.at[gidx_v.at[HALF - 1]],
                              rows2.at[0], gsem).wait()
    plsc.subcore_barrier()
    dw = [pltpu.async_copy(acc_sh.at[pl.ds(s * RPT + k * CHUNK, CHUNK)],
                           acc_hbm.at[c, pl.ds(s * RPT + k * CHUNK, CHUNK)],
                           gsem)
          for k in range(KSLAB)]
    for d in dw:
        d.wait()


def _sc_mesh():
    return plsc.VectorSubcoreMesh(core_axis_name="c", subcore_axis_name="s")


def _sc_degrees(nidx, hidx, eattr):
    return pl.kernel(
        _deg_body,
        compiler_params=pltpu.CompilerParams(needs_layout_passes=False),
        out_type=(jax.ShapeDtypeStruct((NW, NPAD), _f32),
                  jax.ShapeDtypeStruct((NW, NPAD), _f32),
                  jax.ShapeDtypeStruct((NW, NPAD), _f32)),
        mesh=_sc_mesh(),
        scratch_types=[
            pltpu.VMEM((GD, CHUNK), jnp.int32),
            pltpu.VMEM((GD, CHUNK), jnp.int32),
            pltpu.VMEM((NPAD,), _f32),
            pltpu.VMEM((NPAD,), _f32),
            pltpu.VMEM((NPAD,), _f32),
            pltpu.VMEM((NPAD,), _f32),
        ],
    )(nidx, hidx, eattr)


def _sc_prop(src, gidx, sidx):
    return pl.kernel(
        _sc_prop_body,
        compiler_params=pltpu.CompilerParams(needs_layout_passes=False),
        out_type=jax.ShapeDtypeStruct((NC, NPAD, DIN), _f32),
        mesh=_sc_mesh(),
        scratch_types=[
            pltpu.VMEM((HALF, CHUNK), jnp.int32),
            pltpu.VMEM((HALF, CHUNK), jnp.int32),
            pltpu.VMEM((2, CHUNK, DIN), _f32),
            pltpu.VMEM_SHARED((NPAD, DIN), _f32),
            pltpu.SemaphoreType.DMA,
        ],
    )(src, gidx, sidx)


BR = 512


def _tca_body(accp, dp, bp, cp, oute, binv, dinv, sval):
    i = pl.program_id(0)
    ones = jnp.ones((NW, 1), _f32)
    dn = (((0,), (0,)), ((), ()))
    dsum = lax.dot_general(dp[...], ones, dn, preferred_element_type=_f32)
    bsum = lax.dot_general(bp[...], ones, dn, preferred_element_type=_f32)
    csum = lax.dot_general(cp[...], ones, dn, preferred_element_type=_f32)
    rowid = i * BR + lax.broadcasted_iota(jnp.int32, (BR, 1), 0)
    valid = rowid < N
    bi = jnp.where(valid & (bsum > 0), 1.0 / bsum, 0.0)
    di = jnp.where(valid & (dsum > 0), 1.0 / dsum, 0.0)
    oute[...] = bi * (accp[0] + accp[1])
    binv[...] = bi
    dinv[...] = di
    sval[...] = csum * di


def _tc_combine_a(accp, dp, bp, cp):
    return pl.pallas_call(
        _tca_body,
        grid=(NPAD // BR,),
        in_specs=[
            pl.BlockSpec((2, BR, DIN), lambda i: (0, i, 0)),
            pl.BlockSpec((NW, BR), lambda i: (0, i)),
            pl.BlockSpec((NW, BR), lambda i: (0, i)),
            pl.BlockSpec((NW, BR), lambda i: (0, i)),
        ],
        out_specs=[
            pl.BlockSpec((BR, DIN), lambda i: (i, 0)),
            pl.BlockSpec((BR, 1), lambda i: (i, 0)),
            pl.BlockSpec((BR, 1), lambda i: (i, 0)),
            pl.BlockSpec((BR, 1), lambda i: (i, 0)),
        ],
        out_shape=[
            jax.ShapeDtypeStruct((NPAD, DIN), _f32),
            jax.ShapeDtypeStruct((NPAD, 1), _f32),
            jax.ShapeDtypeStruct((NPAD, 1), _f32),
            jax.ShapeDtypeStruct((NPAD, 1), _f32),
        ],
    )(accp, dp, bp, cp)


def _tcc_body(accp, binv, oute):
    oute[...] = binv[...] * (accp[0] + accp[1])


def _tc_combine_c(accp, binv):
    return pl.pallas_call(
        _tcc_body,
        grid=(NPAD // BR,),
        in_specs=[
            pl.BlockSpec((2, BR, DIN), lambda i: (0, i, 0)),
            pl.BlockSpec((BR, 1), lambda i: (i, 0)),
        ],
        out_specs=pl.BlockSpec((BR, DIN), lambda i: (i, 0)),
        out_shape=jax.ShapeDtypeStruct((NPAD, DIN), _f32),
    )(accp, binv)


def _tcb_body(accp, dinv, sval, mask, w1, b1, w2, b2, zout):
    xp = dinv[...] * (accp[0] + accp[1])
    dn = (((1,), (1,)), ((), ()))
    pre = lax.dot_general(xp, w1[...], dn, preferred_element_type=_f32)
    pre = pre + sval[...] * b1[...]
    h = jnp.maximum(pre, 0.0) * mask[...]
    z = lax.dot_general(h, w2[...], dn, preferred_element_type=_f32) + b2[...]
    zout[...] = z


def _tc_mlp(accp, dinv, sval, mask, w1, b1, w2, b2):
    return pl.pallas_call(
        _tcb_body,
        grid=(NPAD // BR,),
        in_specs=[
            pl.BlockSpec((2, BR, DIN), lambda i: (0, i, 0)),
            pl.BlockSpec((BR, 1), lambda i: (i, 0)),
            pl.BlockSpec((BR, 1), lambda i: (i, 0)),
            pl.BlockSpec((BR, DH), lambda i: (i, 0)),
            pl.BlockSpec((DH, DIN), lambda i: (0, 0)),
            pl.BlockSpec((1, DH), lambda i: (0, 0)),
            pl.BlockSpec((DOUT, DH), lambda i: (0, 0)),
            pl.BlockSpec((1, DOUT), lambda i: (0, 0)),
        ],
        out_specs=pl.BlockSpec((BR, DOUT), lambda i: (i, 0)),
        out_shape=jax.ShapeDtypeStruct((NPAD, DOUT), _f32),
    )(accp, dinv, sval, mask, w1, b1, w2, b2)


def _tcd_body(accp, dinv, gamma, beta, yout):
    v = dinv[...] * (accp[0] + accp[1])
    mu = jnp.mean(v, axis=1, keepdims=True)
    d = v - mu
    var = jnp.mean(d * d, axis=1, keepdims=True)
    yout[...] = d * lax.rsqrt(var + EPS) * gamma[...] + beta[...]


def _tc_layernorm(accp, dinv, gamma, beta):
    return pl.pallas_call(
        _tcd_body,
        grid=(NPAD // BR,),
        in_specs=[
            pl.BlockSpec((2, BR, DOUT), lambda i: (0, i, 0)),
            pl.BlockSpec((BR, 1), lambda i: (i, 0)),
            pl.BlockSpec((1, DOUT), lambda i: (0, 0)),
            pl.BlockSpec((1, DOUT), lambda i: (0, 0)),
        ],
        out_specs=pl.BlockSpec((BR, DOUT), lambda i: (i, 0)),
        out_shape=jax.ShapeDtypeStruct((NPAD, DOUT), _f32),
    )(accp, dinv, gamma, beta)


def kernel(x, edge_index, edge_attr, W1, b1, W2, b2, gamma, beta):
    xpad = jnp.zeros((NPAD, DIN), _f32).at[:N].set(x)
    eattr = jnp.zeros((NPAD,), _f32).at[:NHE].set(edge_attr)
    npadinc = IDXROWS * CHUNK - NI
    pad = DUMMY + (jnp.arange(npadinc, dtype=jnp.int32) % (NPAD - N))
    nidx = jnp.concatenate([edge_index[0], pad]).reshape(IDXROWS, CHUNK)
    hidx = jnp.concatenate([edge_index[1], pad]).reshape(IDXROWS, CHUNK)
    keep = jax.random.bernoulli(jax.random.key(42), KEEP, (N, DH))
    mask = jnp.zeros((NPAD, DH), _f32).at[:N].set(
        jnp.where(keep, 1.0 / KEEP, 0.0))

    dpart, bpart, cpart = _sc_degrees(nidx, hidx, eattr)
    acc_e = _sc_prop(xpad, nidx, hidx)
    out_e, binv, dinv, sval = _tc_combine_a(acc_e, dpart, bpart, cpart)
    acc_n = _sc_prop(out_e, hidx, nidx)
    z = _tc_mlp(acc_n, dinv, sval, mask, W1, b1.reshape(1, DH),
                W2, b2.reshape(1, DOUT))

    acc_e2 = _sc_prop(z, nidx, hidx)
    out_e2 = _tc_combine_c(acc_e2, binv)
    acc_n2 = _sc_prop(out_e2, hidx, nidx)
    y = _tc_layernorm(acc_n2, dinv, gamma.reshape(1, DOUT),
                      beta.reshape(1, DOUT))
    return y[:N]


# speedup vs baseline: 1.2855x; 1.0425x over previous
import functools

import jax
import jax.numpy as jnp
from jax import lax
from jax.experimental import pallas as pl
from jax.experimental.pallas import tpu as pltpu
from jax.experimental.pallas import tpu_sc as plsc

N = 10000
NHE = 10000
NI = 320000
DIN = 128
DH = 256
DOUT = 128
EPS = 1e-5
KEEP = 0.7

NPAD = 10240
DUMMY = 10000
NC = 2
NS = 16
NW = NC * NS
CHUNK = 128
GD = 80
TOTCH = NW * GD
NIPAD = TOTCH * CHUNK
G0 = 80
G1 = TOTCH // NS - G0
GMAX = max(G0, G1)
IDXROWS = TOTCH + GMAX
RPT = NPAD // NS
KSLAB = RPT // CHUNK

_f32 = jnp.float32


def _zero_rows_buf(buf):
    zeros16 = jnp.zeros((16,), _f32)

    def body(r, carry):
        for l in range(DIN // 16):
            buf[r, pl.ds(l * 16, 16)] = zeros16
        return carry

    lax.fori_loop(0, CHUNK, body, 0)


def _zero_vec(ref):
    zeros16 = jnp.zeros((16,), _f32)

    def body(i, carry):
        ref[pl.ds(i * 16, 16)] = zeros16
        return carry

    lax.fori_loop(0, NPAD // 16, body, 0)


def _deg_body(nidx_hbm, hidx_hbm, eattr_hbm,
              dp_hbm, bp_hbm, cp_hbm,
              nidx_v, hidx_v, eattr_v, dloc, bloc, cloc):
    c = lax.axis_index("c")
    s = lax.axis_index("s")
    w = s * NC + c
    pltpu.sync_copy(nidx_hbm.at[pl.ds(w * GD, GD)], nidx_v)
    pltpu.sync_copy(hidx_hbm.at[pl.ds(w * GD, GD)], hidx_v)
    pltpu.sync_copy(eattr_hbm, eattr_v)
    _zero_vec(dloc)
    _zero_vec(bloc)
    _zero_vec(cloc)
    ones16 = jnp.ones((16,), _f32)

    def chunk(g, carry):
        for j in range(CHUNK // 16):
            ni = nidx_v[g, pl.ds(j * 16, 16)]
            hi = hidx_v[g, pl.ds(j * 16, 16)]
            wv = plsc.load_gather(eattr_v, [hi])
            plsc.addupdate_scatter(dloc, [ni], wv)
            plsc.addupdate_scatter(bloc, [hi], ones16)
            plsc.addupdate_scatter(cloc, [ni], ones16)
        return carry

    lax.fori_loop(0, GD, chunk, 0)
    pltpu.sync_copy(dloc, dp_hbm.at[w])
    pltpu.sync_copy(bloc, bp_hbm.at[w])
    pltpu.sync_copy(cloc, cp_hbm.at[w])


HALF = GMAX // 2


def _sc_prop_body(src_hbm, gidx_hbm, sidx_hbm, acc_hbm,
                  gidx_v, sidx_v, rows2, acc_sh, gsem):
    c = lax.axis_index("c")
    s = lax.axis_index("s")
    base = (s * NC + c) * GMAX
    _zero_rows_buf(rows2.at[0])
    dz = [pltpu.async_copy(rows2.at[0],
                           acc_sh.at[pl.ds(s * RPT + k * CHUNK, CHUNK)],
                           gsem)
          for k in range(KSLAB)]
    for d in dz:
        d.wait()
    plsc.subcore_barrier()

    for h in range(2):
        pltpu.sync_copy(gidx_hbm.at[pl.ds(base + h * HALF, HALF)], gidx_v)
        pltpu.sync_copy(sidx_hbm.at[pl.ds(base + h * HALF, HALF)], sidx_v)
        pltpu.async_copy(src_hbm.at[gidx_v.at[0]], rows2.at[0], gsem)

        def chunk(g, carry):
            par = lax.rem(g, 2)
            nxt = 1 - par
            gn = jnp.minimum(g + 1, HALF - 1)
            pltpu.async_copy(src_hbm.at[gidx_v.at[gn]], rows2.at[nxt], gsem)
            pltpu.make_async_copy(src_hbm.at[gidx_v.at[g]], rows2.at[par],
                                  gsem).wait()
            pltpu.sync_copy(rows2.at[par], acc_sh.at[sidx_v.at[g]], add=True)
            return carry

        lax.fori_loop(0, HALF, chunk, 0)
        pltpu.make_async_copy(src_hbm.at[gidx_v.at[HALF - 1]],
                              rows2.at[0], gsem).wait()
    plsc.subcore_barrier()
    dw = [pltpu.async_copy(acc_sh.at[pl.ds(s * RPT + k * CHUNK, CHUNK)],
                           acc_hbm.at[c, pl.ds(s * RPT + k * CHUNK, CHUNK)],
                           gsem)
          for k in range(KSLAB)]
    for d in dw:
        d.wait()


def _sc_mesh():
    return plsc.VectorSubcoreMesh(core_axis_name="c", subcore_axis_name="s")


def _sc_degrees(nidx, hidx, eattr):
    return pl.kernel(
        _deg_body,
        compiler_params=pltpu.CompilerParams(needs_layout_passes=False),
        out_type=(jax.ShapeDtypeStruct((NW, NPAD), _f32),
                  jax.ShapeDtypeStruct((NW, NPAD), _f32),
                  jax.ShapeDtypeStruct((NW, NPAD), _f32)),
        mesh=_sc_mesh(),
        scratch_types=[
            pltpu.VMEM((GD, CHUNK), jnp.int32),
            pltpu.VMEM((GD, CHUNK), jnp.int32),
            pltpu.VMEM((NPAD,), _f32),
            pltpu.VMEM((NPAD,), _f32),
            pltpu.VMEM((NPAD,), _f32),
            pltpu.VMEM((NPAD,), _f32),
        ],
    )(nidx, hidx, eattr)


def _sc_prop(src, gidx, sidx):
    return pl.kernel(
        _sc_prop_body,
        compiler_params=pltpu.CompilerParams(needs_layout_passes=False),
        out_type=jax.ShapeDtypeStruct((NC, NPAD, DIN), _f32),
        mesh=_sc_mesh(),
        scratch_types=[
            pltpu.VMEM((HALF, CHUNK), jnp.int32),
            pltpu.VMEM((HALF, CHUNK), jnp.int32),
            pltpu.VMEM((2, CHUNK, DIN), _f32),
            pltpu.VMEM_SHARED((NPAD, DIN), _f32),
            pltpu.SemaphoreType.DMA,
        ],
    )(src, gidx, sidx)


BR = 1024


def _tca_body(accp, dp, bp, cp, oute, binv, dinv, sval):
    i = pl.program_id(0)
    ones = jnp.ones((NW, 1), _f32)
    dn = (((0,), (0,)), ((), ()))
    dsum = lax.dot_general(dp[...], ones, dn, preferred_element_type=_f32)
    bsum = lax.dot_general(bp[...], ones, dn, preferred_element_type=_f32)
    csum = lax.dot_general(cp[...], ones, dn, preferred_element_type=_f32)
    rowid = i * BR + lax.broadcasted_iota(jnp.int32, (BR, 1), 0)
    valid = rowid < N
    bi = jnp.where(valid & (bsum > 0), 1.0 / bsum, 0.0)
    di = jnp.where(valid & (dsum > 0), 1.0 / dsum, 0.0)
    oute[...] = bi * (accp[0] + accp[1])
    binv[...] = bi
    dinv[...] = di
    sval[...] = csum * di


def _tc_combine_a(accp, dp, bp, cp):
    return pl.pallas_call(
        _tca_body,
        grid=(NPAD // BR,),
        in_specs=[
            pl.BlockSpec((2, BR, DIN), lambda i: (0, i, 0)),
            pl.BlockSpec((NW, BR), lambda i: (0, i)),
            pl.BlockSpec((NW, BR), lambda i: (0, i)),
            pl.BlockSpec((NW, BR), lambda i: (0, i)),
        ],
        out_specs=[
            pl.BlockSpec((BR, DIN), lambda i: (i, 0)),
            pl.BlockSpec((BR, 1), lambda i: (i, 0)),
            pl.BlockSpec((BR, 1), lambda i: (i, 0)),
            pl.BlockSpec((BR, 1), lambda i: (i, 0)),
        ],
        out_shape=[
            jax.ShapeDtypeStruct((NPAD, DIN), _f32),
            jax.ShapeDtypeStruct((NPAD, 1), _f32),
            jax.ShapeDtypeStruct((NPAD, 1), _f32),
            jax.ShapeDtypeStruct((NPAD, 1), _f32),
        ],
    )(accp, dp, bp, cp)


def _tcc_body(accp, binv, oute):
    oute[...] = binv[...] * (accp[0] + accp[1])


def _tc_combine_c(accp, binv):
    return pl.pallas_call(
        _tcc_body,
        grid=(NPAD // BR,),
        in_specs=[
            pl.BlockSpec((2, BR, DIN), lambda i: (0, i, 0)),
            pl.BlockSpec((BR, 1), lambda i: (i, 0)),
        ],
        out_specs=pl.BlockSpec((BR, DIN), lambda i: (i, 0)),
        out_shape=jax.ShapeDtypeStruct((NPAD, DIN), _f32),
    )(accp, binv)


def _tcb_body(accp, dinv, sval, mask, w1, b1, w2, b2, zout):
    xp = dinv[...] * (accp[0] + accp[1])
    dn = (((1,), (1,)), ((), ()))
    pre = lax.dot_general(xp, w1[...], dn, preferred_element_type=_f32)
    pre = pre + sval[...] * b1[...]
    h = jnp.maximum(pre, 0.0) * mask[...]
    z = lax.dot_general(h, w2[...], dn, preferred_element_type=_f32) + b2[...]
    zout[...] = z


def _tc_mlp(accp, dinv, sval, mask, w1, b1, w2, b2):
    return pl.pallas_call(
        _tcb_body,
        grid=(NPAD // BR,),
        in_specs=[
            pl.BlockSpec((2, BR, DIN), lambda i: (0, i, 0)),
            pl.BlockSpec((BR, 1), lambda i: (i, 0)),
            pl.BlockSpec((BR, 1), lambda i: (i, 0)),
            pl.BlockSpec((BR, DH), lambda i: (i, 0)),
            pl.BlockSpec((DH, DIN), lambda i: (0, 0)),
            pl.BlockSpec((1, DH), lambda i: (0, 0)),
            pl.BlockSpec((DOUT, DH), lambda i: (0, 0)),
            pl.BlockSpec((1, DOUT), lambda i: (0, 0)),
        ],
        out_specs=pl.BlockSpec((BR, DOUT), lambda i: (i, 0)),
        out_shape=jax.ShapeDtypeStruct((NPAD, DOUT), _f32),
    )(accp, dinv, sval, mask, w1, b1, w2, b2)


def _tcd_body(accp, dinv, gamma, beta, yout):
    v = dinv[...] * (accp[0] + accp[1])
    mu = jnp.mean(v, axis=1, keepdims=True)
    d = v - mu
    var = jnp.mean(d * d, axis=1, keepdims=True)
    yout[...] = d * lax.rsqrt(var + EPS) * gamma[...] + beta[...]


def _tc_layernorm(accp, dinv, gamma, beta):
    return pl.pallas_call(
        _tcd_body,
        grid=(NPAD // BR,),
        in_specs=[
            pl.BlockSpec((2, BR, DOUT), lambda i: (0, i, 0)),
            pl.BlockSpec((BR, 1), lambda i: (i, 0)),
            pl.BlockSpec((1, DOUT), lambda i: (0, 0)),
            pl.BlockSpec((1, DOUT), lambda i: (0, 0)),
        ],
        out_specs=pl.BlockSpec((BR, DOUT), lambda i: (i, 0)),
        out_shape=jax.ShapeDtypeStruct((NPAD, DOUT), _f32),
    )(accp, dinv, gamma, beta)


def kernel(x, edge_index, edge_attr, W1, b1, W2, b2, gamma, beta):
    xpad = jnp.zeros((NPAD, DIN), _f32).at[:N].set(x)
    eattr = jnp.zeros((NPAD,), _f32).at[:NHE].set(edge_attr)
    npadinc = IDXROWS * CHUNK - NI
    pad = DUMMY + (jnp.arange(npadinc, dtype=jnp.int32) % (NPAD - N))
    nidx = jnp.concatenate([edge_index[0], pad]).reshape(IDXROWS, CHUNK)
    hidx = jnp.concatenate([edge_index[1], pad]).reshape(IDXROWS, CHUNK)
    keep = jax.random.bernoulli(jax.random.key(42), KEEP, (N, DH))
    mask = jnp.zeros((NPAD, DH), _f32).at[:N].set(
        jnp.where(keep, 1.0 / KEEP, 0.0))

    dpart, bpart, cpart = _sc_degrees(nidx, hidx, eattr)
    acc_e = _sc_prop(xpad, nidx, hidx)
    out_e, binv, dinv, sval = _tc_combine_a(acc_e, dpart, bpart, cpart)
    acc_n = _sc_prop(out_e, hidx, nidx)
    z = _tc_mlp(acc_n, dinv, sval, mask, W1, b1.reshape(1, DH),
                W2, b2.reshape(1, DOUT))

    acc_e2 = _sc_prop(z, nidx, hidx)
    out_e2 = _tc_combine_c(acc_e2, binv)
    acc_n2 = _sc_prop(out_e2, hidx, nidx)
    y = _tc_layernorm(acc_n2, dinv, gamma.reshape(1, DOUT),
                      beta.reshape(1, DOUT))
    return y[:N]


# speedup vs baseline: 1.3057x; 1.0157x over previous
import functools

import jax
import jax.numpy as jnp
from jax import lax
from jax.experimental import pallas as pl
from jax.experimental.pallas import tpu as pltpu
from jax.experimental.pallas import tpu_sc as plsc

N = 10000
NHE = 10000
NI = 320000
DIN = 128
DH = 256
DOUT = 128
EPS = 1e-5
KEEP = 0.7

NPAD = 10240
DUMMY = 10000
NC = 2
NS = 16
NW = NC * NS
CHUNK = 128
GD = 80
TOTCH = NW * GD
NIPAD = TOTCH * CHUNK
G0 = 80
G1 = TOTCH // NS - G0
GMAX = max(G0, G1)
IDXROWS = TOTCH + GMAX
RPT = NPAD // NS
KSLAB = RPT // CHUNK

_f32 = jnp.float32


def _zero_rows_buf(buf):
    zeros16 = jnp.zeros((16,), _f32)

    def body(r, carry):
        for l in range(DIN // 16):
            buf[r, pl.ds(l * 16, 16)] = zeros16
        return carry

    lax.fori_loop(0, CHUNK, body, 0)


def _zero_vec(ref):
    zeros16 = jnp.zeros((16,), _f32)

    def body(i, carry):
        ref[pl.ds(i * 16, 16)] = zeros16
        return carry

    lax.fori_loop(0, NPAD // 16, body, 0)


def _deg_body(nidx_hbm, hidx_hbm, eattr_hbm,
              dp_hbm, bp_hbm, cp_hbm,
              nidx_v, hidx_v, eattr_v, dloc, bloc, cloc):
    c = lax.axis_index("c")
    s = lax.axis_index("s")
    w = s * NC + c
    pltpu.sync_copy(nidx_hbm.at[pl.ds(w * GD, GD)], nidx_v)
    pltpu.sync_copy(hidx_hbm.at[pl.ds(w * GD, GD)], hidx_v)
    pltpu.sync_copy(eattr_hbm, eattr_v)
    _zero_vec(dloc)
    _zero_vec(bloc)
    _zero_vec(cloc)
    ones16 = jnp.ones((16,), _f32)

    def chunk(g, carry):
        for j in range(CHUNK // 16):
            ni = nidx_v[g, pl.ds(j * 16, 16)]
            hi = hidx_v[g, pl.ds(j * 16, 16)]
            wv = plsc.load_gather(eattr_v, [hi])
            plsc.addupdate_scatter(dloc, [ni], wv)
            plsc.addupdate_scatter(bloc, [hi], ones16)
            plsc.addupdate_scatter(cloc, [ni], ones16)
        return carry

    lax.fori_loop(0, GD, chunk, 0)
    pltpu.sync_copy(dloc, dp_hbm.at[w])
    pltpu.sync_copy(bloc, bp_hbm.at[w])
    pltpu.sync_copy(cloc, cp_hbm.at[w])


HALF = GMAX // 2


def _sc_prop_body(src_hbm, gidx_hbm, sidx_hbm, acc_hbm,
                  gidx_v, sidx_v, rows2, acc_sh, gsem):
    c = lax.axis_index("c")
    s = lax.axis_index("s")
    base = (s * NC + c) * GMAX
    _zero_rows_buf(rows2.at[0])
    dz = [pltpu.async_copy(rows2.at[0],
                           acc_sh.at[pl.ds(s * RPT + k * CHUNK, CHUNK)],
                           gsem)
          for k in range(KSLAB)]
    for d in dz:
        d.wait()
    plsc.subcore_barrier()

    for h in range(2):
        pltpu.sync_copy(gidx_hbm.at[pl.ds(base + h * HALF, HALF)], gidx_v)
        pltpu.sync_copy(sidx_hbm.at[pl.ds(base + h * HALF, HALF)], sidx_v)
        pltpu.async_copy(src_hbm.at[gidx_v.at[0]], rows2.at[0], gsem)

        def chunk(g, carry):
            par = lax.rem(g, 2)
            nxt = 1 - par
            gn = jnp.minimum(g + 1, HALF - 1)
            pltpu.async_copy(src_hbm.at[gidx_v.at[gn]], rows2.at[nxt], gsem)
            pltpu.make_async_copy(src_hbm.at[gidx_v.at[g]], rows2.at[par],
                                  gsem).wait()
            pltpu.sync_copy(rows2.at[par], acc_sh.at[sidx_v.at[g]], add=True)
            return carry

        lax.fori_loop(0, HALF, chunk, 0)
        pltpu.make_async_copy(src_hbm.at[gidx_v.at[HALF - 1]],
                              rows2.at[0], gsem).wait()
    plsc.subcore_barrier()
    dw = [pltpu.async_copy(acc_sh.at[pl.ds(s * RPT + k * CHUNK, CHUNK)],
                           acc_hbm.at[c, pl.ds(s * RPT + k * CHUNK, CHUNK)],
                           gsem)
          for k in range(KSLAB)]
    for d in dw:
        d.wait()


def _sc_mesh():
    return plsc.VectorSubcoreMesh(core_axis_name="c", subcore_axis_name="s")


def _sc_degrees(nidx, hidx, eattr):
    return pl.kernel(
        _deg_body,
        compiler_params=pltpu.CompilerParams(needs_layout_passes=False),
        out_type=(jax.ShapeDtypeStruct((NW, NPAD), _f32),
                  jax.ShapeDtypeStruct((NW, NPAD), _f32),
                  jax.ShapeDtypeStruct((NW, NPAD), _f32)),
        mesh=_sc_mesh(),
        scratch_types=[
            pltpu.VMEM((GD, CHUNK), jnp.int32),
            pltpu.VMEM((GD, CHUNK), jnp.int32),
            pltpu.VMEM((NPAD,), _f32),
            pltpu.VMEM((NPAD,), _f32),
            pltpu.VMEM((NPAD,), _f32),
            pltpu.VMEM((NPAD,), _f32),
        ],
    )(nidx, hidx, eattr)


def _sc_prop(src, gidx, sidx):
    return pl.kernel(
        _sc_prop_body,
        compiler_params=pltpu.CompilerParams(needs_layout_passes=False),
        out_type=jax.ShapeDtypeStruct((NC, NPAD, DIN), _f32),
        mesh=_sc_mesh(),
        scratch_types=[
            pltpu.VMEM((HALF, CHUNK), jnp.int32),
            pltpu.VMEM((HALF, CHUNK), jnp.int32),
            pltpu.VMEM((2, CHUNK, DIN), _f32),
            pltpu.VMEM_SHARED((NPAD, DIN), _f32),
            pltpu.SemaphoreType.DMA,
        ],
    )(src, gidx, sidx)


BR = 2048


def _tca_body(accp, dp, bp, cp, oute, binv, dinv, sval):
    i = pl.program_id(0)
    ones = jnp.ones((NW, 1), _f32)
    dn = (((0,), (0,)), ((), ()))
    dsum = lax.dot_general(dp[...], ones, dn, preferred_element_type=_f32)
    bsum = lax.dot_general(bp[...], ones, dn, preferred_element_type=_f32)
    csum = lax.dot_general(cp[...], ones, dn, preferred_element_type=_f32)
    rowid = i * BR + lax.broadcasted_iota(jnp.int32, (BR, 1), 0)
    valid = rowid < N
    bi = jnp.where(valid & (bsum > 0), 1.0 / bsum, 0.0)
    di = jnp.where(valid & (dsum > 0), 1.0 / dsum, 0.0)
    oute[...] = bi * (accp[0] + accp[1])
    binv[...] = bi
    dinv[...] = di
    sval[...] = csum * di


def _tc_combine_a(accp, dp, bp, cp):
    return pl.pallas_call(
        _tca_body,
        grid=(NPAD // BR,),
        in_specs=[
            pl.BlockSpec((2, BR, DIN), lambda i: (0, i, 0)),
            pl.BlockSpec((NW, BR), lambda i: (0, i)),
            pl.BlockSpec((NW, BR), lambda i: (0, i)),
            pl.BlockSpec((NW, BR), lambda i: (0, i)),
        ],
        out_specs=[
            pl.BlockSpec((BR, DIN), lambda i: (i, 0)),
            pl.BlockSpec((BR, 1), lambda i: (i, 0)),
            pl.BlockSpec((BR, 1), lambda i: (i, 0)),
            pl.BlockSpec((BR, 1), lambda i: (i, 0)),
        ],
        out_shape=[
            jax.ShapeDtypeStruct((NPAD, DIN), _f32),
            jax.ShapeDtypeStruct((NPAD, 1), _f32),
            jax.ShapeDtypeStruct((NPAD, 1), _f32),
            jax.ShapeDtypeStruct((NPAD, 1), _f32),
        ],
    )(accp, dp, bp, cp)


def _tcc_body(accp, binv, oute):
    oute[...] = binv[...] * (accp[0] + accp[1])


def _tc_combine_c(accp, binv):
    return pl.pallas_call(
        _tcc_body,
        grid=(NPAD // BR,),
        in_specs=[
            pl.BlockSpec((2, BR, DIN), lambda i: (0, i, 0)),
            pl.BlockSpec((BR, 1), lambda i: (i, 0)),
        ],
        out_specs=pl.BlockSpec((BR, DIN), lambda i: (i, 0)),
        out_shape=jax.ShapeDtypeStruct((NPAD, DIN), _f32),
    )(accp, binv)


def _tcb_body(accp, dinv, sval, mask, w1, b1, w2, b2, zout):
    xp = dinv[...] * (accp[0] + accp[1])
    dn = (((1,), (1,)), ((), ()))
    pre = lax.dot_general(xp, w1[...], dn, preferred_element_type=_f32)
    pre = pre + sval[...] * b1[...]
    h = jnp.maximum(pre, 0.0) * mask[...]
    z = lax.dot_general(h, w2[...], dn, preferred_element_type=_f32) + b2[...]
    zout[...] = z


def _tc_mlp(accp, dinv, sval, mask, w1, b1, w2, b2):
    return pl.pallas_call(
        _tcb_body,
        grid=(NPAD // BR,),
        in_specs=[
            pl.BlockSpec((2, BR, DIN), lambda i: (0, i, 0)),
            pl.BlockSpec((BR, 1), lambda i: (i, 0)),
            pl.BlockSpec((BR, 1), lambda i: (i, 0)),
            pl.BlockSpec((BR, DH), lambda i: (i, 0)),
            pl.BlockSpec((DH, DIN), lambda i: (0, 0)),
            pl.BlockSpec((1, DH), lambda i: (0, 0)),
            pl.BlockSpec((DOUT, DH), lambda i: (0, 0)),
            pl.BlockSpec((1, DOUT), lambda i: (0, 0)),
        ],
        out_specs=pl.BlockSpec((BR, DOUT), lambda i: (i, 0)),
        out_shape=jax.ShapeDtypeStruct((NPAD, DOUT), _f32),
    )(accp, dinv, sval, mask, w1, b1, w2, b2)


def _tcd_body(accp, dinv, gamma, beta, yout):
    v = dinv[...] * (accp[0] + accp[1])
    mu = jnp.mean(v, axis=1, keepdims=True)
    d = v - mu
    var = jnp.mean(d * d, axis=1, keepdims=True)
    yout[...] = d * lax.rsqrt(var + EPS) * gamma[...] + beta[...]


def _tc_layernorm(accp, dinv, gamma, beta):
    return pl.pallas_call(
        _tcd_body,
        grid=(NPAD // BR,),
        in_specs=[
            pl.BlockSpec((2, BR, DOUT), lambda i: (0, i, 0)),
            pl.BlockSpec((BR, 1), lambda i: (i, 0)),
            pl.BlockSpec((1, DOUT), lambda i: (0, 0)),
            pl.BlockSpec((1, DOUT), lambda i: (0, 0)),
        ],
        out_specs=pl.BlockSpec((BR, DOUT), lambda i: (i, 0)),
        out_shape=jax.ShapeDtypeStruct((NPAD, DOUT), _f32),
    )(accp, dinv, gamma, beta)


def kernel(x, edge_index, edge_attr, W1, b1, W2, b2, gamma, beta):
    xpad = jnp.zeros((NPAD, DIN), _f32).at[:N].set(x)
    eattr = jnp.zeros((NPAD,), _f32).at[:NHE].set(edge_attr)
    npadinc = IDXROWS * CHUNK - NI
    pad = DUMMY + (jnp.arange(npadinc, dtype=jnp.int32) % (NPAD - N))
    nidx = jnp.concatenate([edge_index[0], pad]).reshape(IDXROWS, CHUNK)
    hidx = jnp.concatenate([edge_index[1], pad]).reshape(IDXROWS, CHUNK)
    keep = jax.random.bernoulli(jax.random.key(42), KEEP, (N, DH))
    mask = jnp.zeros((NPAD, DH), _f32).at[:N].set(
        jnp.where(keep, 1.0 / KEEP, 0.0))

    dpart, bpart, cpart = _sc_degrees(nidx, hidx, eattr)
    acc_e = _sc_prop(xpad, nidx, hidx)
    out_e, binv, dinv, sval = _tc_combine_a(acc_e, dpart, bpart, cpart)
    acc_n = _sc_prop(out_e, hidx, nidx)
    z = _tc_mlp(acc_n, dinv, sval, mask, W1, b1.reshape(1, DH),
                W2, b2.reshape(1, DOUT))

    acc_e2 = _sc_prop(z, nidx, hidx)
    out_e2 = _tc_combine_c(acc_e2, binv)
    acc_n2 = _sc_prop(out_e2, hidx, nidx)
    y = _tc_layernorm(acc_n2, dinv, gamma.reshape(1, DOUT),
                      beta.reshape(1, DOUT))
    return y[:N]


# speedup vs baseline: 1.3125x; 1.0052x over previous
import functools

import jax
import jax.numpy as jnp
from jax import lax
from jax.experimental import pallas as pl
from jax.experimental.pallas import tpu as pltpu
from jax.experimental.pallas import tpu_sc as plsc

N = 10000
NHE = 10000
NI = 320000
DIN = 128
DH = 256
DOUT = 128
EPS = 1e-5
KEEP = 0.7

NPAD = 10240
DUMMY = 10000
NC = 2
NS = 16
NW = NC * NS
CHUNK = 128
GD = 80
TOTCH = NW * GD
NIPAD = TOTCH * CHUNK
G0 = 80
G1 = TOTCH // NS - G0
GMAX = max(G0, G1)
IDXROWS = TOTCH + GMAX
RPT = NPAD // NS
KSLAB = RPT // CHUNK

_f32 = jnp.float32


def _zero_rows_buf(buf):
    zeros16 = jnp.zeros((16,), _f32)

    def body(r, carry):
        for l in range(DIN // 16):
            buf[r, pl.ds(l * 16, 16)] = zeros16
        return carry

    lax.fori_loop(0, CHUNK, body, 0)


def _zero_vec(ref):
    zeros16 = jnp.zeros((16,), _f32)

    def body(i, carry):
        ref[pl.ds(i * 16, 16)] = zeros16
        return carry

    lax.fori_loop(0, NPAD // 16, body, 0)


def _deg_body(nidx_hbm, hidx_hbm, eattr_hbm,
              dp_hbm, bp_hbm, cp_hbm,
              nidx_v, hidx_v, eattr_v, dloc, bloc, cloc):
    c = lax.axis_index("c")
    s = lax.axis_index("s")
    w = s * NC + c
    pltpu.sync_copy(nidx_hbm.at[pl.ds(w * GD, GD)], nidx_v)
    pltpu.sync_copy(hidx_hbm.at[pl.ds(w * GD, GD)], hidx_v)
    pltpu.sync_copy(eattr_hbm, eattr_v)
    _zero_vec(dloc)
    _zero_vec(bloc)
    _zero_vec(cloc)
    ones16 = jnp.ones((16,), _f32)

    def chunk(g, carry):
        for j in range(CHUNK // 16):
            ni = nidx_v[g, pl.ds(j * 16, 16)]
            hi = hidx_v[g, pl.ds(j * 16, 16)]
            wv = plsc.load_gather(eattr_v, [hi])
            plsc.addupdate_scatter(dloc, [ni], wv)
            plsc.addupdate_scatter(bloc, [hi], ones16)
            plsc.addupdate_scatter(cloc, [ni], ones16)
        return carry

    lax.fori_loop(0, GD, chunk, 0)
    pltpu.sync_copy(dloc, dp_hbm.at[w])
    pltpu.sync_copy(bloc, bp_hbm.at[w])
    pltpu.sync_copy(cloc, cp_hbm.at[w])


HALF = GMAX // 2


def _sc_prop_body(src_hbm, gidx_hbm, sidx_hbm, acc_hbm,
                  gidx_v, sidx_v, rows2, acc_sh, gsem):
    c = lax.axis_index("c")
    s = lax.axis_index("s")
    base = (s * NC + c) * GMAX
    _zero_rows_buf(rows2.at[0])
    dz = [pltpu.async_copy(rows2.at[0],
                           acc_sh.at[pl.ds(s * RPT + k * CHUNK, CHUNK)],
                           gsem)
          for k in range(KSLAB)]
    for d in dz:
        d.wait()
    plsc.subcore_barrier()

    for h in range(2):
        pltpu.sync_copy(gidx_hbm.at[pl.ds(base + h * HALF, HALF)], gidx_v)
        pltpu.sync_copy(sidx_hbm.at[pl.ds(base + h * HALF, HALF)], sidx_v)
        pltpu.async_copy(src_hbm.at[gidx_v.at[0]], rows2.at[0], gsem)

        def chunk(g, carry):
            par = lax.rem(g, 2)
            nxt = 1 - par
            gn = jnp.minimum(g + 1, HALF - 1)
            pltpu.async_copy(src_hbm.at[gidx_v.at[gn]], rows2.at[nxt], gsem)
            pltpu.make_async_copy(src_hbm.at[gidx_v.at[g]], rows2.at[par],
                                  gsem).wait()
            pltpu.sync_copy(rows2.at[par], acc_sh.at[sidx_v.at[g]], add=True)
            return carry

        lax.fori_loop(0, HALF, chunk, 0)
        pltpu.make_async_copy(src_hbm.at[gidx_v.at[HALF - 1]],
                              rows2.at[0], gsem).wait()
    plsc.subcore_barrier()
    dw = [pltpu.async_copy(acc_sh.at[pl.ds(s * RPT + k * CHUNK, CHUNK)],
                           acc_hbm.at[c, pl.ds(s * RPT + k * CHUNK, CHUNK)],
                           gsem)
          for k in range(KSLAB)]
    for d in dw:
        d.wait()


def _sc_mesh():
    return plsc.VectorSubcoreMesh(core_axis_name="c", subcore_axis_name="s")


def _sc_degrees(nidx, hidx, eattr):
    return pl.kernel(
        _deg_body,
        compiler_params=pltpu.CompilerParams(needs_layout_passes=False),
        out_type=(jax.ShapeDtypeStruct((NW, NPAD), _f32),
                  jax.ShapeDtypeStruct((NW, NPAD), _f32),
                  jax.ShapeDtypeStruct((NW, NPAD), _f32)),
        mesh=_sc_mesh(),
        scratch_types=[
            pltpu.VMEM((GD, CHUNK), jnp.int32),
            pltpu.VMEM((GD, CHUNK), jnp.int32),
            pltpu.VMEM((NPAD,), _f32),
            pltpu.VMEM((NPAD,), _f32),
            pltpu.VMEM((NPAD,), _f32),
            pltpu.VMEM((NPAD,), _f32),
        ],
    )(nidx, hidx, eattr)


def _sc_prop(src, gidx, sidx):
    return pl.kernel(
        _sc_prop_body,
        compiler_params=pltpu.CompilerParams(needs_layout_passes=False),
        out_type=jax.ShapeDtypeStruct((NC, NPAD, DIN), _f32),
        mesh=_sc_mesh(),
        scratch_types=[
            pltpu.VMEM((HALF, CHUNK), jnp.int32),
            pltpu.VMEM((HALF, CHUNK), jnp.int32),
            pltpu.VMEM((2, CHUNK, DIN), _f32),
            pltpu.VMEM_SHARED((NPAD, DIN), _f32),
            pltpu.SemaphoreType.DMA,
        ],
    )(src, gidx, sidx)


BR = 2560


def _tca_body(accp, dp, bp, cp, oute, binv, dinv, sval):
    i = pl.program_id(0)
    ones = jnp.ones((NW, 1), _f32)
    dn = (((0,), (0,)), ((), ()))
    dsum = lax.dot_general(dp[...], ones, dn, preferred_element_type=_f32)
    bsum = lax.dot_general(bp[...], ones, dn, preferred_element_type=_f32)
    csum = lax.dot_general(cp[...], ones, dn, preferred_element_type=_f32)
    rowid = i * BR + lax.broadcasted_iota(jnp.int32, (BR, 1), 0)
    valid = rowid < N
    bi = jnp.where(valid & (bsum > 0), 1.0 / bsum, 0.0)
    di = jnp.where(valid & (dsum > 0), 1.0 / dsum, 0.0)
    oute[...] = bi * (accp[0] + accp[1])
    binv[...] = bi
    dinv[...] = di
    sval[...] = csum * di


def _tc_combine_a(accp, dp, bp, cp):
    return pl.pallas_call(
        _tca_body,
        grid=(NPAD // BR,),
        in_specs=[
            pl.BlockSpec((2, BR, DIN), lambda i: (0, i, 0)),
            pl.BlockSpec((NW, BR), lambda i: (0, i)),
            pl.BlockSpec((NW, BR), lambda i: (0, i)),
            pl.BlockSpec((NW, BR), lambda i: (0, i)),
        ],
        out_specs=[
            pl.BlockSpec((BR, DIN), lambda i: (i, 0)),
            pl.BlockSpec((BR, 1), lambda i: (i, 0)),
            pl.BlockSpec((BR, 1), lambda i: (i, 0)),
            pl.BlockSpec((BR, 1), lambda i: (i, 0)),
        ],
        out_shape=[
            jax.ShapeDtypeStruct((NPAD, DIN), _f32),
            jax.ShapeDtypeStruct((NPAD, 1), _f32),
            jax.ShapeDtypeStruct((NPAD, 1), _f32),
            jax.ShapeDtypeStruct((NPAD, 1), _f32),
        ],
    )(accp, dp, bp, cp)


def _tcc_body(accp, binv, oute):
    oute[...] = binv[...] * (accp[0] + accp[1])


def _tc_combine_c(accp, binv):
    return pl.pallas_call(
        _tcc_body,
        grid=(NPAD // BR,),
        in_specs=[
            pl.BlockSpec((2, BR, DIN), lambda i: (0, i, 0)),
            pl.BlockSpec((BR, 1), lambda i: (i, 0)),
        ],
        out_specs=pl.BlockSpec((BR, DIN), lambda i: (i, 0)),
        out_shape=jax.ShapeDtypeStruct((NPAD, DIN), _f32),
    )(accp, binv)


def _tcb_body(accp, dinv, sval, mask, w1, b1, w2, b2, zout):
    xp = dinv[...] * (accp[0] + accp[1])
    dn = (((1,), (1,)), ((), ()))
    pre = lax.dot_general(xp, w1[...], dn, preferred_element_type=_f32)
    pre = pre + sval[...] * b1[...]
    h = jnp.maximum(pre, 0.0) * mask[...]
    z = lax.dot_general(h, w2[...], dn, preferred_element_type=_f32) + b2[...]
    zout[...] = z


def _tc_mlp(accp, dinv, sval, mask, w1, b1, w2, b2):
    return pl.pallas_call(
        _tcb_body,
        grid=(NPAD // BR,),
        in_specs=[
            pl.BlockSpec((2, BR, DIN), lambda i: (0, i, 0)),
            pl.BlockSpec((BR, 1), lambda i: (i, 0)),
            pl.BlockSpec((BR, 1), lambda i: (i, 0)),
            pl.BlockSpec((BR, DH), lambda i: (i, 0)),
            pl.BlockSpec((DH, DIN), lambda i: (0, 0)),
            pl.BlockSpec((1, DH), lambda i: (0, 0)),
            pl.BlockSpec((DOUT, DH), lambda i: (0, 0)),
            pl.BlockSpec((1, DOUT), lambda i: (0, 0)),
        ],
        out_specs=pl.BlockSpec((BR, DOUT), lambda i: (i, 0)),
        out_shape=jax.ShapeDtypeStruct((NPAD, DOUT), _f32),
    )(accp, dinv, sval, mask, w1, b1, w2, b2)


def _tcd_body(accp, dinv, gamma, beta, yout):
    v = dinv[...] * (accp[0] + accp[1])
    mu = jnp.mean(v, axis=1, keepdims=True)
    d = v - mu
    var = jnp.mean(d * d, axis=1, keepdims=True)
    yout[...] = d * lax.rsqrt(var + EPS) * gamma[...] + beta[...]


def _tc_layernorm(accp, dinv, gamma, beta):
    return pl.pallas_call(
        _tcd_body,
        grid=(NPAD // BR,),
        in_specs=[
            pl.BlockSpec((2, BR, DOUT), lambda i: (0, i, 0)),
            pl.BlockSpec((BR, 1), lambda i: (i, 0)),
            pl.BlockSpec((1, DOUT), lambda i: (0, 0)),
            pl.BlockSpec((1, DOUT), lambda i: (0, 0)),
        ],
        out_specs=pl.BlockSpec((BR, DOUT), lambda i: (i, 0)),
        out_shape=jax.ShapeDtypeStruct((NPAD, DOUT), _f32),
    )(accp, dinv, gamma, beta)


def kernel(x, edge_index, edge_attr, W1, b1, W2, b2, gamma, beta):
    xpad = jnp.zeros((NPAD, DIN), _f32).at[:N].set(x)
    eattr = jnp.zeros((NPAD,), _f32).at[:NHE].set(edge_attr)
    npadinc = IDXROWS * CHUNK - NI
    pad = DUMMY + (jnp.arange(npadinc, dtype=jnp.int32) % (NPAD - N))
    nidx = jnp.concatenate([edge_index[0], pad]).reshape(IDXROWS, CHUNK)
    hidx = jnp.concatenate([edge_index[1], pad]).reshape(IDXROWS, CHUNK)
    keep = jax.random.bernoulli(jax.random.key(42), KEEP, (N, DH))
    mask = jnp.zeros((NPAD, DH), _f32).at[:N].set(
        jnp.where(keep, 1.0 / KEEP, 0.0))

    dpart, bpart, cpart = _sc_degrees(nidx, hidx, eattr)
    acc_e = _sc_prop(xpad, nidx, hidx)
    out_e, binv, dinv, sval = _tc_combine_a(acc_e, dpart, bpart, cpart)
    acc_n = _sc_prop(out_e, hidx, nidx)
    z = _tc_mlp(acc_n, dinv, sval, mask, W1, b1.reshape(1, DH),
                W2, b2.reshape(1, DOUT))

    acc_e2 = _sc_prop(z, nidx, hidx)
    out_e2 = _tc_combine_c(acc_e2, binv)
    acc_n2 = _sc_prop(out_e2, hidx, nidx)
    y = _tc_layernorm(acc_n2, dinv, gamma.reshape(1, DOUT),
                      beta.reshape(1, DOUT))
    return y[:N]


# speedup vs baseline: 1.3129x; 1.0003x over previous
import jax
import jax.numpy as jnp
from jax import lax
from jax.experimental import pallas as pl
from jax.experimental.pallas import tpu as pltpu
from jax.experimental.pallas import tpu_sc as plsc

N = 10000
NHE = 10000
NI = 320000
DIN = 128
DH = 256
DOUT = 128
EPS = 1e-5
KEEP = 0.7

NPAD = 10240
DUMMY = 10000
NC = 2
NS = 16
NW = NC * NS
CHUNK = 128
GD = 80
TOTCH = NW * GD
NIPAD = TOTCH * CHUNK
G0 = 80
G1 = TOTCH // NS - G0
GMAX = max(G0, G1)
IDXROWS = TOTCH + GMAX
RPT = NPAD // NS
KSLAB = RPT // CHUNK

_f32 = jnp.float32


def _zero_rows_buf(buf):
    zeros16 = jnp.zeros((16,), _f32)

    def body(r, carry):
        for l in range(DIN // 16):
            buf[r, pl.ds(l * 16, 16)] = zeros16
        return carry

    lax.fori_loop(0, CHUNK, body, 0)


def _zero_vec(ref):
    zeros16 = jnp.zeros((16,), _f32)

    def body(i, carry):
        ref[pl.ds(i * 16, 16)] = zeros16
        return carry

    lax.fori_loop(0, NPAD // 16, body, 0)


def _deg_body(nidx_hbm, hidx_hbm, eattr_hbm,
              dp_hbm, bp_hbm, cp_hbm,
              nidx_v, hidx_v, eattr_v, dloc, bloc, cloc):
    c = lax.axis_index("c")
    s = lax.axis_index("s")
    w = s * NC + c
    pltpu.sync_copy(nidx_hbm.at[pl.ds(w * GD, GD)], nidx_v)
    pltpu.sync_copy(hidx_hbm.at[pl.ds(w * GD, GD)], hidx_v)
    pltpu.sync_copy(eattr_hbm, eattr_v)
    _zero_vec(dloc)
    _zero_vec(bloc)
    _zero_vec(cloc)
    ones16 = jnp.ones((16,), _f32)

    def chunk(g, carry):
        for j in range(CHUNK // 16):
            ni = nidx_v[g, pl.ds(j * 16, 16)]
            hi = hidx_v[g, pl.ds(j * 16, 16)]
            wv = plsc.load_gather(eattr_v, [hi])
            plsc.addupdate_scatter(dloc, [ni], wv)
            plsc.addupdate_scatter(bloc, [hi], ones16)
            plsc.addupdate_scatter(cloc, [ni], ones16)
        return carry

    lax.fori_loop(0, GD, chunk, 0)
    pltpu.sync_copy(dloc, dp_hbm.at[w])
    pltpu.sync_copy(bloc, bp_hbm.at[w])
    pltpu.sync_copy(cloc, cp_hbm.at[w])


HALF = GMAX // 2


def _sc_prop_body(src_hbm, gidx_hbm, sidx_hbm, acc_hbm,
                  gidx_v, sidx_v, rows2, acc_sh, gsem):
    c = lax.axis_index("c")
    s = lax.axis_index("s")
    base = (s * NC + c) * GMAX
    _zero_rows_buf(rows2.at[0])
    dz = [pltpu.async_copy(rows2.at[0],
                           acc_sh.at[pl.ds(s * RPT + k * CHUNK, CHUNK)],
                           gsem)
          for k in range(KSLAB)]
    for d in dz:
        d.wait()
    plsc.subcore_barrier()

    for h in range(2):
        pltpu.sync_copy(gidx_hbm.at[pl.ds(base + h * HALF, HALF)], gidx_v)
        pltpu.sync_copy(sidx_hbm.at[pl.ds(base + h * HALF, HALF)], sidx_v)
        pltpu.async_copy(src_hbm.at[gidx_v.at[0]], rows2.at[0], gsem)

        def chunk(g, carry):
            par = lax.rem(g, 2)
            nxt = 1 - par
            gn = jnp.minimum(g + 1, HALF - 1)
            pltpu.async_copy(src_hbm.at[gidx_v.at[gn]], rows2.at[nxt], gsem)
            pltpu.make_async_copy(src_hbm.at[gidx_v.at[g]], rows2.at[par],
                                  gsem).wait()
            pltpu.sync_copy(rows2.at[par], acc_sh.at[sidx_v.at[g]], add=True)
            return carry

        lax.fori_loop(0, HALF, chunk, 0)
        pltpu.make_async_copy(src_hbm.at[gidx_v.at[HALF - 1]],
                              rows2.at[0], gsem).wait()
    plsc.subcore_barrier()
    dw = [pltpu.async_copy(acc_sh.at[pl.ds(s * RPT + k * CHUNK, CHUNK)],
                           acc_hbm.at[c, pl.ds(s * RPT + k * CHUNK, CHUNK)],
                           gsem)
          for k in range(KSLAB)]
    for d in dw:
        d.wait()


def _sc_mesh():
    return plsc.VectorSubcoreMesh(core_axis_name="c", subcore_axis_name="s")


def _sc_degrees(nidx, hidx, eattr):
    return pl.kernel(
        _deg_body,
        compiler_params=pltpu.CompilerParams(needs_layout_passes=False),
        out_type=(jax.ShapeDtypeStruct((NW, NPAD), _f32),
                  jax.ShapeDtypeStruct((NW, NPAD), _f32),
                  jax.ShapeDtypeStruct((NW, NPAD), _f32)),
        mesh=_sc_mesh(),
        scratch_types=[
            pltpu.VMEM((GD, CHUNK), jnp.int32),
            pltpu.VMEM((GD, CHUNK), jnp.int32),
            pltpu.VMEM((NPAD,), _f32),
            pltpu.VMEM((NPAD,), _f32),
            pltpu.VMEM((NPAD,), _f32),
            pltpu.VMEM((NPAD,), _f32),
        ],
    )(nidx, hidx, eattr)


def _sc_prop(src, gidx, sidx):
    return pl.kernel(
        _sc_prop_body,
        compiler_params=pltpu.CompilerParams(needs_layout_passes=False),
        out_type=jax.ShapeDtypeStruct((NC, NPAD, DIN), _f32),
        mesh=_sc_mesh(),
        scratch_types=[
            pltpu.VMEM((HALF, CHUNK), jnp.int32),
            pltpu.VMEM((HALF, CHUNK), jnp.int32),
            pltpu.VMEM((2, CHUNK, DIN), _f32),
            pltpu.VMEM_SHARED((NPAD, DIN), _f32),
            pltpu.SemaphoreType.DMA,
        ],
    )(src, gidx, sidx)


BR = 2560


def _tca_body(accp, dp, bp, cp, oute, binv, dinv, sval):
    i = pl.program_id(0)
    ones = jnp.ones((NW, 1), _f32)
    dn = (((0,), (0,)), ((), ()))
    dsum = lax.dot_general(dp[...], ones, dn, preferred_element_type=_f32)
    bsum = lax.dot_general(bp[...], ones, dn, preferred_element_type=_f32)
    csum = lax.dot_general(cp[...], ones, dn, preferred_element_type=_f32)
    rowid = i * BR + lax.broadcasted_iota(jnp.int32, (BR, 1), 0)
    valid = rowid < N
    bi = jnp.where(valid & (bsum > 0), 1.0 / bsum, 0.0)
    di = jnp.where(valid & (dsum > 0), 1.0 / dsum, 0.0)
    oute[...] = bi * (accp[0] + accp[1])
    binv[...] = bi
    dinv[...] = di
    sval[...] = csum * di


def _tc_combine_a(accp, dp, bp, cp):
    return pl.pallas_call(
        _tca_body,
        grid=(NPAD // BR,),
        in_specs=[
            pl.BlockSpec((2, BR, DIN), lambda i: (0, i, 0)),
            pl.BlockSpec((NW, BR), lambda i: (0, i)),
            pl.BlockSpec((NW, BR), lambda i: (0, i)),
            pl.BlockSpec((NW, BR), lambda i: (0, i)),
        ],
        out_specs=[
            pl.BlockSpec((BR, DIN), lambda i: (i, 0)),
            pl.BlockSpec((BR, 1), lambda i: (i, 0)),
            pl.BlockSpec((BR, 1), lambda i: (i, 0)),
            pl.BlockSpec((BR, 1), lambda i: (i, 0)),
        ],
        out_shape=[
            jax.ShapeDtypeStruct((NPAD, DIN), _f32),
            jax.ShapeDtypeStruct((NPAD, 1), _f32),
            jax.ShapeDtypeStruct((NPAD, 1), _f32),
            jax.ShapeDtypeStruct((NPAD, 1), _f32),
        ],
    )(accp, dp, bp, cp)


def _tcc_body(accp, binv, oute):
    oute[...] = binv[...] * (accp[0] + accp[1])


def _tc_combine_c(accp, binv):
    return pl.pallas_call(
        _tcc_body,
        grid=(NPAD // BR,),
        in_specs=[
            pl.BlockSpec((2, BR, DIN), lambda i: (0, i, 0)),
            pl.BlockSpec((BR, 1), lambda i: (i, 0)),
        ],
        out_specs=pl.BlockSpec((BR, DIN), lambda i: (i, 0)),
        out_shape=jax.ShapeDtypeStruct((NPAD, DIN), _f32),
    )(accp, binv)


def _tcb_body(accp, dinv, sval, mask, w1, b1, w2, b2, zout):
    xp = dinv[...] * (accp[0] + accp[1])
    dn = (((1,), (1,)), ((), ()))
    pre = lax.dot_general(xp, w1[...], dn, preferred_element_type=_f32)
    pre = pre + sval[...] * b1[...]
    h = jnp.maximum(pre, 0.0) * mask[...]
    z = lax.dot_general(h, w2[...], dn, preferred_element_type=_f32) + b2[...]
    zout[...] = z


def _tc_mlp(accp, dinv, sval, mask, w1, b1, w2, b2):
    return pl.pallas_call(
        _tcb_body,
        grid=(NPAD // BR,),
        in_specs=[
            pl.BlockSpec((2, BR, DIN), lambda i: (0, i, 0)),
            pl.BlockSpec((BR, 1), lambda i: (i, 0)),
            pl.BlockSpec((BR, 1), lambda i: (i, 0)),
            pl.BlockSpec((BR, DH), lambda i: (i, 0)),
            pl.BlockSpec((DH, DIN), lambda i: (0, 0)),
            pl.BlockSpec((1, DH), lambda i: (0, 0)),
            pl.BlockSpec((DOUT, DH), lambda i: (0, 0)),
            pl.BlockSpec((1, DOUT), lambda i: (0, 0)),
        ],
        out_specs=pl.BlockSpec((BR, DOUT), lambda i: (i, 0)),
        out_shape=jax.ShapeDtypeStruct((NPAD, DOUT), _f32),
    )(accp, dinv, sval, mask, w1, b1, w2, b2)


def _tcd_body(accp, dinv, gamma, beta, yout):
    v = dinv[...] * (accp[0] + accp[1])
    mu = jnp.mean(v, axis=1, keepdims=True)
    d = v - mu
    var = jnp.mean(d * d, axis=1, keepdims=True)
    yout[...] = d * lax.rsqrt(var + EPS) * gamma[...] + beta[...]


def _tc_layernorm(accp, dinv, gamma, beta):
    return pl.pallas_call(
        _tcd_body,
        grid=(NPAD // BR,),
        in_specs=[
            pl.BlockSpec((2, BR, DOUT), lambda i: (0, i, 0)),
            pl.BlockSpec((BR, 1), lambda i: (i, 0)),
            pl.BlockSpec((1, DOUT), lambda i: (0, 0)),
            pl.BlockSpec((1, DOUT), lambda i: (0, 0)),
        ],
        out_specs=pl.BlockSpec((BR, DOUT), lambda i: (i, 0)),
        out_shape=jax.ShapeDtypeStruct((NPAD, DOUT), _f32),
    )(accp, dinv, gamma, beta)


def kernel(x, edge_index, edge_attr, W1, b1, W2, b2, gamma, beta):
    xpad = jnp.zeros((NPAD, DIN), _f32).at[:N].set(x)
    eattr = jnp.zeros((NPAD,), _f32).at[:NHE].set(edge_attr)
    npadinc = IDXROWS * CHUNK - NI
    pad = DUMMY + (jnp.arange(npadinc, dtype=jnp.int32) % (NPAD - N))
    nidx = jnp.concatenate([edge_index[0], pad]).reshape(IDXROWS, CHUNK)
    hidx = jnp.concatenate([edge_index[1], pad]).reshape(IDXROWS, CHUNK)
    keep = jax.random.bernoulli(jax.random.key(42), KEEP, (N, DH))
    mask = jnp.zeros((NPAD, DH), _f32).at[:N].set(
        jnp.where(keep, 1.0 / KEEP, 0.0))

    dpart, bpart, cpart = _sc_degrees(nidx, hidx, eattr)
    acc_e = _sc_prop(xpad, nidx, hidx)
    out_e, binv, dinv, sval = _tc_combine_a(acc_e, dpart, bpart, cpart)
    acc_n = _sc_prop(out_e, hidx, nidx)
    z = _tc_mlp(acc_n, dinv, sval, mask, W1, b1.reshape(1, DH),
                W2, b2.reshape(1, DOUT))

    acc_e2 = _sc_prop(z, nidx, hidx)
    out_e2 = _tc_combine_c(acc_e2, binv)
    acc_n2 = _sc_prop(out_e2, hidx, nidx)
    y = _tc_layernorm(acc_n2, dinv, gamma.reshape(1, DOUT),
                      beta.reshape(1, DOUT))
    return y[:N]
